# baseline (speedup 1.0000x reference)
import functools

import jax
import jax.numpy as jnp
from jax import lax
from jax.experimental import pallas as pl
from jax.experimental.pallas import tpu as pltpu
from jax.experimental.pallas import tpu_sc as plsc

_B, _V, _H, _W = 8, 4, 512, 512
_N = 2048
_ROWS = _B * _V
_PIX = _H * _W
_RCH = 32
_CHUNK = _RCH * _W
_NCHUNK = _PIX // _CHUNK
_L = 16
_UNROLL = 16
_PCH = _N // _V


def _body(seg_hbm, p2p_hbm, out_hbm,
          seg_v, p2p_v, votes_v, counts_v,
          votes_sh, counts_sh, vtmp, ctmp, acc_v, nv_v, sem0, sem1):
    cid = lax.axis_index("c")
    sid = lax.axis_index("s")
    wid = cid * 16 + sid
    sems = (sem0, sem1)

    zeros16 = jnp.zeros((_L,), jnp.float32)

    def zero_body(i, carry):
        votes_v[pl.ds(i * _L, _L)] = zeros16
        counts_v[pl.ds(i * _L, _L)] = zeros16
        return carry

    lax.fori_loop(0, _N // _L, zero_body, 0)

    ones16 = jnp.ones((_L,), jnp.float32)

    def issue(ci, buf):
        r0 = ci * _RCH
        pltpu.async_copy(seg_hbm.at[wid, pl.ds(r0, _RCH), :],
                         seg_v.at[buf], sems[buf])
        pltpu.async_copy(p2p_hbm.at[wid, pl.ds(r0, _RCH), :],
                         p2p_v.at[buf], sems[buf])

    def wait(buf):
        pltpu.make_async_copy(seg_hbm.at[0, pl.ds(0, _RCH), :],
                              seg_v.at[buf], sems[buf]).wait()
        pltpu.make_async_copy(p2p_hbm.at[0, pl.ds(0, _RCH), :],
                              p2p_v.at[buf], sems[buf]).wait()

    issue(0, 0)

    def outer_body(oi, carry):
        for b in (0, 1):
            ci = oi * 2 + b

            @pl.when(ci + 1 < _NCHUNK)
            def _():
                issue(ci + 1, 1 - b)

            wait(b)

            @plsc.parallel_loop(0, _CHUNK // _L, unroll=_UNROLL)
            def _pix(i):
                r = i >> 5
                c = (i & 31) * _L
                idx = p2p_v[b, r, pl.ds(c, _L)]
                val = seg_v[b, r, pl.ds(c, _L)]
                m = idx >= 0
                plsc.addupdate_scatter(votes_v, [idx], val, mask=m)
                plsc.addupdate_scatter(counts_v, [idx], ones16, mask=m)
        return carry

    lax.fori_loop(0, _NCHUNK // 2, outer_body, 0)

    pltpu.sync_copy(votes_v, votes_sh.at[sid])
    pltpu.sync_copy(counts_v, counts_sh.at[sid])
    plsc.subcore_barrier()

    blocal = sid // 4
    bat = cid * 4 + blocal
    p0 = (sid % 4) * _PCH
    for v in range(_V):
        pltpu.sync_copy(votes_sh.at[blocal * 4 + v, pl.ds(p0, _PCH)],
                        vtmp.at[v])
        pltpu.sync_copy(counts_sh.at[blocal * 4 + v, pl.ds(p0, _PCH)],
                        ctmp.at[v])

    @plsc.parallel_loop(0, _PCH // _L, unroll=4)
    def _cmb(i):
        o = i * _L
        acc = jnp.zeros((_L,), jnp.float32)
        nv = jnp.zeros((_L,), jnp.float32)
        for v in range(_V):
            vv = vtmp[v, pl.ds(o, _L)]
            cc = ctmp[v, pl.ds(o, _L)]
            acc = acc + vv / jnp.maximum(cc, 1.0)
            nv = nv + jnp.where(cc > 0, 1.0, 0.0).astype(jnp.float32)
        acc_v[pl.ds(o, _L)] = acc / jnp.maximum(nv, 1.0)

    pltpu.sync_copy(acc_v, out_hbm.at[bat, pl.ds(p0, _PCH)])


_agg = functools.partial(
    pl.kernel,
    out_type=jax.ShapeDtypeStruct((_B, _N), jnp.float32),
    mesh=plsc.VectorSubcoreMesh(core_axis_name="c", subcore_axis_name="s"),
    scratch_types=[pltpu.VMEM((2, _RCH, _W), jnp.float32),
                   pltpu.VMEM((2, _RCH, _W), jnp.int32),
                   pltpu.VMEM((_N,), jnp.float32),
                   pltpu.VMEM((_N,), jnp.float32),
                   pltpu.VMEM_SHARED((16, _N), jnp.float32),
                   pltpu.VMEM_SHARED((16, _N), jnp.float32),
                   pltpu.VMEM((_V, _PCH), jnp.float32),
                   pltpu.VMEM((_V, _PCH), jnp.float32),
                   pltpu.VMEM((_PCH,), jnp.float32),
                   pltpu.VMEM((_PCH,), jnp.float32),
                   pltpu.SemaphoreType.DMA,
                   pltpu.SemaphoreType.DMA],
    compiler_params=pltpu.CompilerParams(needs_layout_passes=False,
                                         use_tc_tiling_on_sc=True),
)(_body)


def kernel(seg_maps, p2p_maps):
    seg = seg_maps.reshape(_ROWS, _H, _W)
    p2p = p2p_maps.reshape(_ROWS, _H, _W)
    return _agg(seg, p2p)


# speedup vs baseline: 1.2710x; 1.2710x over previous
import functools

import jax
import jax.numpy as jnp
from jax import lax
from jax.experimental import pallas as pl
from jax.experimental.pallas import tpu as pltpu
from jax.experimental.pallas import tpu_sc as plsc

_B, _V, _H, _W = 8, 4, 512, 512
_N = 2048
_ROWS = _B * _V
_PIX = _H * _W
_RCH = 16
_CHUNK = _RCH * _W
_NCHUNK = _PIX // _CHUNK
_L = 16
_UNROLL = 8
_PCH = _N // _V


def _body(seg_hbm, p2p_hbm, out_hbm,
          seg_v, p2p_v, votes_r, counts_r, votes_v, counts_v,
          votes_sh, counts_sh, vtmp, ctmp, acc_v, nv_v, sem0, sem1):
    cid = lax.axis_index("c")
    sid = lax.axis_index("s")
    wid = cid * 16 + sid
    sems = (sem0, sem1)

    zeros16 = jnp.zeros((_L,), jnp.float32)

    @plsc.parallel_loop(0, _N, unroll=8)
    def _zero(n):
        votes_r[pl.ds(n * _L, _L)] = zeros16
        counts_r[pl.ds(n * _L, _L)] = zeros16

    ones16 = jnp.ones((_L,), jnp.float32)
    lane16 = lax.iota(jnp.int32, _L)

    def issue(ci, buf):
        r0 = ci * _RCH
        pltpu.async_copy(seg_hbm.at[wid, pl.ds(r0, _RCH), :],
                         seg_v.at[buf], sems[buf])
        pltpu.async_copy(p2p_hbm.at[wid, pl.ds(r0, _RCH), :],
                         p2p_v.at[buf], sems[buf])

    def wait(buf):
        pltpu.make_async_copy(seg_hbm.at[0, pl.ds(0, _RCH), :],
                              seg_v.at[buf], sems[buf]).wait()
        pltpu.make_async_copy(p2p_hbm.at[0, pl.ds(0, _RCH), :],
                              p2p_v.at[buf], sems[buf]).wait()

    issue(0, 0)

    def outer_body(oi, carry):
        for b in (0, 1):
            ci = oi * 2 + b

            @pl.when(ci + 1 < _NCHUNK)
            def _():
                issue(ci + 1, 1 - b)

            wait(b)

            @plsc.parallel_loop(0, _CHUNK // _L, unroll=_UNROLL)
            def _pix(i):
                r = i >> 5
                c = (i & 31) * _L
                idx = p2p_v[b, r, pl.ds(c, _L)]
                val = seg_v[b, r, pl.ds(c, _L)]
                m = idx >= 0
                slot = (idx << 4) + lane16
                plsc.addupdate_scatter(votes_r, [slot], val, mask=m)
                plsc.addupdate_scatter(counts_r, [slot], ones16, mask=m)
        return carry

    lax.fori_loop(0, _NCHUNK // 2, outer_body, 0)

    @plsc.parallel_loop(0, _N // _L, unroll=2)
    def _merge(g):
        accv = zeros16
        accc = zeros16
        for j in range(_L):
            n = g * _L + j
            sv = jnp.sum(votes_r[pl.ds(n * _L, _L)])
            sc = jnp.sum(counts_r[pl.ds(n * _L, _L)])
            sel = lane16 == j
            accv = jnp.where(sel, sv, accv)
            accc = jnp.where(sel, sc, accc)
        votes_v[pl.ds(g * _L, _L)] = accv
        counts_v[pl.ds(g * _L, _L)] = accc

    pltpu.sync_copy(votes_v, votes_sh.at[sid])
    pltpu.sync_copy(counts_v, counts_sh.at[sid])
    plsc.subcore_barrier()

    blocal = sid // 4
    bat = cid * 4 + blocal
    p0 = (sid % 4) * _PCH
    for v in range(_V):
        pltpu.sync_copy(votes_sh.at[blocal * 4 + v, pl.ds(p0, _PCH)],
                        vtmp.at[v])
        pltpu.sync_copy(counts_sh.at[blocal * 4 + v, pl.ds(p0, _PCH)],
                        ctmp.at[v])

    @plsc.parallel_loop(0, _PCH // _L, unroll=4)
    def _cmb(i):
        o = i * _L
        acc = jnp.zeros((_L,), jnp.float32)
        nv = jnp.zeros((_L,), jnp.float32)
        for v in range(_V):
            vv = vtmp[v, pl.ds(o, _L)]
            cc = ctmp[v, pl.ds(o, _L)]
            acc = acc + vv / jnp.maximum(cc, 1.0)
            nv = nv + jnp.where(cc > 0, 1.0, 0.0).astype(jnp.float32)
        acc_v[pl.ds(o, _L)] = acc / jnp.maximum(nv, 1.0)

    pltpu.sync_copy(acc_v, out_hbm.at[bat, pl.ds(p0, _PCH)])


_agg = functools.partial(
    pl.kernel,
    out_type=jax.ShapeDtypeStruct((_B, _N), jnp.float32),
    mesh=plsc.VectorSubcoreMesh(core_axis_name="c", subcore_axis_name="s"),
    scratch_types=[pltpu.VMEM((2, _RCH, _W), jnp.float32),
                   pltpu.VMEM((2, _RCH, _W), jnp.int32),
                   pltpu.VMEM((_N * _L,), jnp.float32),
                   pltpu.VMEM((_N * _L,), jnp.float32),
                   pltpu.VMEM((_N,), jnp.float32),
                   pltpu.VMEM((_N,), jnp.float32),
                   pltpu.VMEM_SHARED((16, _N), jnp.float32),
                   pltpu.VMEM_SHARED((16, _N), jnp.float32),
                   pltpu.VMEM((_V, _PCH), jnp.float32),
                   pltpu.VMEM((_V, _PCH), jnp.float32),
                   pltpu.VMEM((_PCH,), jnp.float32),
                   pltpu.VMEM((_PCH,), jnp.float32),
                   pltpu.SemaphoreType.DMA,
                   pltpu.SemaphoreType.DMA],
    compiler_params=pltpu.CompilerParams(needs_layout_passes=False,
                                         use_tc_tiling_on_sc=True),
)(_body)


def kernel(seg_maps, p2p_maps):
    seg = seg_maps.reshape(_ROWS, _H, _W)
    p2p = p2p_maps.reshape(_ROWS, _H, _W)
    return _agg(seg, p2p)


# speedup vs baseline: 1.2951x; 1.0189x over previous
import functools

import jax
import jax.numpy as jnp
from jax import lax
from jax.experimental import pallas as pl
from jax.experimental.pallas import tpu as pltpu
from jax.experimental.pallas import tpu_sc as plsc

_B, _V, _H, _W = 8, 4, 512, 512
_N = 2048
_ROWS = _B * _V
_PIX = _H * _W
_RCH = 16
_CHUNK = _RCH * _W
_NCHUNK = _PIX // _CHUNK
_L = 16
_UNROLL = 16
_PCH = _N // _V


def _body(seg_hbm, p2p_hbm, out_hbm,
          seg_v, p2p_v, votes_r, counts_r, votes_v, counts_v,
          votes_sh, counts_sh, vtmp, ctmp, acc_v, nv_v, sem0, sem1):
    cid = lax.axis_index("c")
    sid = lax.axis_index("s")
    wid = cid * 16 + sid
    sems = (sem0, sem1)

    zeros16 = jnp.zeros((_L,), jnp.float32)
    ones16 = jnp.ones((_L,), jnp.float32)
    lane16 = lax.iota(jnp.int32, _L)

    def issue(ci, buf):
        r0 = ci * _RCH
        pltpu.async_copy(seg_hbm.at[wid, pl.ds(r0, _RCH), :],
                         seg_v.at[buf], sems[buf])
        pltpu.async_copy(p2p_hbm.at[wid, pl.ds(r0, _RCH), :],
                         p2p_v.at[buf], sems[buf])

    def wait(buf):
        pltpu.make_async_copy(seg_hbm.at[0, pl.ds(0, _RCH), :],
                              seg_v.at[buf], sems[buf]).wait()
        pltpu.make_async_copy(p2p_hbm.at[0, pl.ds(0, _RCH), :],
                              p2p_v.at[buf], sems[buf]).wait()

    issue(0, 0)

    @plsc.parallel_loop(0, _N, unroll=8)
    def _zero(n):
        votes_r[pl.ds(n * _L, _L)] = zeros16
        counts_r[pl.ds(n * _L, _L)] = zeros16

    def outer_body(oi, carry):
        for b in (0, 1):
            ci = oi * 2 + b

            @pl.when(ci + 1 < _NCHUNK)
            def _():
                issue(ci + 1, 1 - b)

            wait(b)

            @plsc.parallel_loop(0, _CHUNK // _L, unroll=_UNROLL)
            def _pix(i):
                r = i >> 5
                c = (i & 31) * _L
                idx = p2p_v[b, r, pl.ds(c, _L)]
                val = seg_v[b, r, pl.ds(c, _L)]
                m = idx >= 0
                slot = (idx << 4) + lane16
                plsc.addupdate_scatter(votes_r, [slot], val, mask=m)
                plsc.addupdate_scatter(counts_r, [slot], ones16, mask=m)
        return carry

    lax.fori_loop(0, _NCHUNK // 2, outer_body, 0)

    @plsc.parallel_loop(0, _N // _L, unroll=2)
    def _merge(g):
        accv = zeros16
        accc = zeros16
        for j in range(_L):
            n = g * _L + j
            sv = jnp.sum(votes_r[pl.ds(n * _L, _L)])
            sc = jnp.sum(counts_r[pl.ds(n * _L, _L)])
            sel = lane16 == j
            accv = jnp.where(sel, sv, accv)
            accc = jnp.where(sel, sc, accc)
        votes_v[pl.ds(g * _L, _L)] = accv
        counts_v[pl.ds(g * _L, _L)] = accc

    pltpu.sync_copy(votes_v, votes_sh.at[sid])
    pltpu.sync_copy(counts_v, counts_sh.at[sid])
    plsc.subcore_barrier()

    blocal = sid // 4
    bat = cid * 4 + blocal
    p0 = (sid % 4) * _PCH
    for v in range(_V):
        pltpu.sync_copy(votes_sh.at[blocal * 4 + v, pl.ds(p0, _PCH)],
                        vtmp.at[v])
        pltpu.sync_copy(counts_sh.at[blocal * 4 + v, pl.ds(p0, _PCH)],
                        ctmp.at[v])

    @plsc.parallel_loop(0, _PCH // _L, unroll=4)
    def _cmb(i):
        o = i * _L
        acc = jnp.zeros((_L,), jnp.float32)
        nv = jnp.zeros((_L,), jnp.float32)
        for v in range(_V):
            vv = vtmp[v, pl.ds(o, _L)]
            cc = ctmp[v, pl.ds(o, _L)]
            acc = acc + vv / jnp.maximum(cc, 1.0)
            nv = nv + jnp.where(cc > 0, 1.0, 0.0).astype(jnp.float32)
        acc_v[pl.ds(o, _L)] = acc / jnp.maximum(nv, 1.0)

    pltpu.sync_copy(acc_v, out_hbm.at[bat, pl.ds(p0, _PCH)])


_agg = functools.partial(
    pl.kernel,
    out_type=jax.ShapeDtypeStruct((_B, _N), jnp.float32),
    mesh=plsc.VectorSubcoreMesh(core_axis_name="c", subcore_axis_name="s"),
    scratch_types=[pltpu.VMEM((2, _RCH, _W), jnp.float32),
                   pltpu.VMEM((2, _RCH, _W), jnp.int32),
                   pltpu.VMEM((_N * _L,), jnp.float32),
                   pltpu.VMEM((_N * _L,), jnp.float32),
                   pltpu.VMEM((_N,), jnp.float32),
                   pltpu.VMEM((_N,), jnp.float32),
                   pltpu.VMEM_SHARED((16, _N), jnp.float32),
                   pltpu.VMEM_SHARED((16, _N), jnp.float32),
                   pltpu.VMEM((_V, _PCH), jnp.float32),
                   pltpu.VMEM((_V, _PCH), jnp.float32),
                   pltpu.VMEM((_PCH,), jnp.float32),
                   pltpu.VMEM((_PCH,), jnp.float32),
                   pltpu.SemaphoreType.DMA,
                   pltpu.SemaphoreType.DMA],
    compiler_params=pltpu.CompilerParams(needs_layout_passes=False,
                                         use_tc_tiling_on_sc=True),
)(_body)


def kernel(seg_maps, p2p_maps):
    seg = seg_maps.reshape(_ROWS, _H, _W)
    p2p = p2p_maps.reshape(_ROWS, _H, _W)
    return _agg(seg, p2p)


# speedup vs baseline: 1.2963x; 1.0010x over previous
import functools

import jax
import jax.numpy as jnp
from jax import lax
from jax.experimental import pallas as pl
from jax.experimental.pallas import tpu as pltpu
from jax.experimental.pallas import tpu_sc as plsc

_B, _V, _H, _W = 8, 4, 512, 512
_N = 2048
_ROWS = _B * _V
_PIX = _H * _W
_RCH = 16
_CHUNK = _RCH * _W
_NCHUNK = _PIX // _CHUNK
_L = 16
_UNROLL = 16
_PCH = _N // _V


def _body(seg_hbm, p2p_hbm, out_hbm,
          seg_v, p2p_v, votes_r, counts_r, votes_v, counts_v,
          votes_sh, counts_sh, vtmp, ctmp, acc_v, nv_v, sem0, sem1):
    cid = lax.axis_index("c")
    sid = lax.axis_index("s")
    wid = cid * 16 + sid
    sems = (sem0, sem1)

    zeros16 = jnp.zeros((_L,), jnp.float32)
    ones16 = jnp.ones((_L,), jnp.float32)
    lane16 = lax.iota(jnp.int32, _L)

    _HR = _RCH // 2

    def issue(ci, buf):
        r0 = ci * _RCH
        for h in (0, 1):
            pltpu.async_copy(
                seg_hbm.at[wid, pl.ds(r0 + h * _HR, _HR), :],
                seg_v.at[buf, pl.ds(h * _HR, _HR), :], sems[buf])
            pltpu.async_copy(
                p2p_hbm.at[wid, pl.ds(r0 + h * _HR, _HR), :],
                p2p_v.at[buf, pl.ds(h * _HR, _HR), :], sems[buf])

    def wait(buf):
        for h in (0, 1):
            pltpu.make_async_copy(
                seg_hbm.at[0, pl.ds(0, _HR), :],
                seg_v.at[buf, pl.ds(h * _HR, _HR), :], sems[buf]).wait()
            pltpu.make_async_copy(
                p2p_hbm.at[0, pl.ds(0, _HR), :],
                p2p_v.at[buf, pl.ds(h * _HR, _HR), :], sems[buf]).wait()

    issue(0, 0)

    @plsc.parallel_loop(0, _N, unroll=8)
    def _zero(n):
        votes_r[pl.ds(n * _L, _L)] = zeros16
        counts_r[pl.ds(n * _L, _L)] = zeros16

    def outer_body(oi, carry):
        for b in (0, 1):
            ci = oi * 2 + b

            @pl.when(ci + 1 < _NCHUNK)
            def _():
                issue(ci + 1, 1 - b)

            wait(b)

            @plsc.parallel_loop(0, _CHUNK // _L, unroll=_UNROLL)
            def _pix(i):
                r = i >> 5
                c = (i & 31) * _L
                idx = p2p_v[b, r, pl.ds(c, _L)]
                val = seg_v[b, r, pl.ds(c, _L)]
                m = idx >= 0
                slot = (idx << 4) + lane16
                plsc.addupdate_scatter(votes_r, [slot], val, mask=m)
                plsc.addupdate_scatter(counts_r, [slot], ones16, mask=m)
        return carry

    lax.fori_loop(0, _NCHUNK // 2, outer_body, 0)

    @plsc.parallel_loop(0, _N // _L, unroll=2)
    def _merge(g):
        accv = zeros16
        accc = zeros16
        for j in range(_L):
            n = g * _L + j
            sv = jnp.sum(votes_r[pl.ds(n * _L, _L)])
            sc = jnp.sum(counts_r[pl.ds(n * _L, _L)])
            sel = lane16 == j
            accv = jnp.where(sel, sv, accv)
            accc = jnp.where(sel, sc, accc)
        votes_v[pl.ds(g * _L, _L)] = accv
        counts_v[pl.ds(g * _L, _L)] = accc

    pltpu.sync_copy(votes_v, votes_sh.at[sid])
    pltpu.sync_copy(counts_v, counts_sh.at[sid])
    plsc.subcore_barrier()

    blocal = sid // 4
    bat = cid * 4 + blocal
    p0 = (sid % 4) * _PCH
    for v in range(_V):
        pltpu.sync_copy(votes_sh.at[blocal * 4 + v, pl.ds(p0, _PCH)],
                        vtmp.at[v])
        pltpu.sync_copy(counts_sh.at[blocal * 4 + v, pl.ds(p0, _PCH)],
                        ctmp.at[v])

    @plsc.parallel_loop(0, _PCH // _L, unroll=4)
    def _cmb(i):
        o = i * _L
        acc = jnp.zeros((_L,), jnp.float32)
        nv = jnp.zeros((_L,), jnp.float32)
        for v in range(_V):
            vv = vtmp[v, pl.ds(o, _L)]
            cc = ctmp[v, pl.ds(o, _L)]
            acc = acc + vv / jnp.maximum(cc, 1.0)
            nv = nv + jnp.where(cc > 0, 1.0, 0.0).astype(jnp.float32)
        acc_v[pl.ds(o, _L)] = acc / jnp.maximum(nv, 1.0)

    pltpu.sync_copy(acc_v, out_hbm.at[bat, pl.ds(p0, _PCH)])


_agg = functools.partial(
    pl.kernel,
    out_type=jax.ShapeDtypeStruct((_B, _N), jnp.float32),
    mesh=plsc.VectorSubcoreMesh(core_axis_name="c", subcore_axis_name="s"),
    scratch_types=[pltpu.VMEM((2, _RCH, _W), jnp.float32),
                   pltpu.VMEM((2, _RCH, _W), jnp.int32),
                   pltpu.VMEM((_N * _L,), jnp.float32),
                   pltpu.VMEM((_N * _L,), jnp.float32),
                   pltpu.VMEM((_N,), jnp.float32),
                   pltpu.VMEM((_N,), jnp.float32),
                   pltpu.VMEM_SHARED((16, _N), jnp.float32),
                   pltpu.VMEM_SHARED((16, _N), jnp.float32),
                   pltpu.VMEM((_V, _PCH), jnp.float32),
                   pltpu.VMEM((_V, _PCH), jnp.float32),
                   pltpu.VMEM((_PCH,), jnp.float32),
                   pltpu.VMEM((_PCH,), jnp.float32),
                   pltpu.SemaphoreType.DMA,
                   pltpu.SemaphoreType.DMA],
    compiler_params=pltpu.CompilerParams(needs_layout_passes=False,
                                         use_tc_tiling_on_sc=True),
)(_body)


def kernel(seg_maps, p2p_maps):
    seg = seg_maps.reshape(_ROWS, _H, _W)
    p2p = p2p_maps.reshape(_ROWS, _H, _W)
    return _agg(seg, p2p)
